# Initial kernel scaffold; baseline (speedup 1.0000x reference)
import jax
import jax.numpy as jnp
from jax.experimental import pallas as pl


def kernel(chunked_states, boundary_mask, boundary_prob):
    raise NotImplementedError("write your pallas kernel here")


# speedup vs baseline: 20.7736x; 20.7736x over previous
import functools

import jax
import jax.numpy as jnp
from jax import lax
from jax.experimental import pallas as pl
from jax.experimental.pallas import tpu as pltpu
from jax.experimental.pallas import tpu_sc as plsc

B = 8
L = 2048
M = 1024
D = 2304

_MESH = plsc.VectorSubcoreMesh(core_axis_name="c", subcore_axis_name="s")
_NC = 2
_NW = 32


def _tc_cumsum_body(m_ref, cum_ref, nt_ref):
    i0 = lax.broadcasted_iota(jnp.int32, (128, 128), 0)
    i1 = lax.broadcasted_iota(jnp.int32, (128, 128), 1)
    U = (i0 <= i1).astype(jnp.float32)
    carry = jnp.zeros((B, 1), jnp.float32)
    for j in range(L // 128):
        blk = m_ref[:, j * 128 : (j + 1) * 128]
        loc = (
            lax.dot_general(
                blk, U, (((1,), (0,)), ((), ())),
                preferred_element_type=jnp.float32,
                precision=lax.Precision.HIGHEST,
            )
            + carry
        )
        cum_ref[:, j * 128 : (j + 1) * 128] = loc.astype(jnp.int32)
        carry = loc[:, 127:128]
    nt_ref[...] = jnp.broadcast_to(carry, (B, 16))


def _tc_cumsum(mask_f32):
    return pl.pallas_call(
        _tc_cumsum_body,
        out_shape=(
            jax.ShapeDtypeStruct((B, L), jnp.int32),
            jax.ShapeDtypeStruct((B, 16), jnp.float32),
        ),
    )(mask_f32)


def _sc_index_body(
    mask_hbm, p_hbm, cum_hbm, nt_hbm, pch_hbm, gidx_hbm,
    mask_v, p_v, cum_v, nt_v, pch_v, gidx_v,
):
    wid = lax.axis_index("s") * _NC + lax.axis_index("c")

    @pl.when(wid < B)
    def _():
        b = wid
        pltpu.sync_copy(mask_hbm.at[b], mask_v)
        pltpu.sync_copy(p_hbm.at[b], p_v)
        pltpu.sync_copy(cum_hbm.at[b], cum_v)
        pltpu.sync_copy(nt_hbm.at[b], nt_v)
        nT = nt_v[pl.ds(0, 16)].astype(jnp.int32)

        def body(i, _):
            mvec = mask_v[pl.ds(i * 16, 16)]
            pvec = p_v[pl.ds(i * 16, 16)]
            cumT = cum_v[pl.ds(i * 16, 16)]
            lidx = i * 16 + lax.iota(jnp.int32, 16)
            rank = jnp.where(mvec > 0.5, cumT - 1, nT + lidx - cumT)
            pc = jnp.clip(pvec, 1e-4, 1.0 - 1e-4)
            plsc.store_scatter(pch_v, [rank], pc)
            gidx_v[pl.ds(i * 16, 16)] = b * M + jnp.clip(cumT - 1, 0, M - 1)
            return _

        lax.fori_loop(0, L // 16, body, jnp.int32(0))
        pltpu.sync_copy(pch_v.at[pl.ds(0, M)], pch_hbm.at[b])
        pltpu.sync_copy(gidx_v, gidx_hbm.at[b])


_sc_index = functools.partial(
    pl.kernel,
    out_type=(
        jax.ShapeDtypeStruct((B, M), jnp.float32),
        jax.ShapeDtypeStruct((B, L), jnp.int32),
    ),
    mesh=_MESH,
    scratch_types=[
        pltpu.VMEM((L,), jnp.float32),
        pltpu.VMEM((L,), jnp.float32),
        pltpu.VMEM((L,), jnp.int32),
        pltpu.VMEM((16,), jnp.float32),
        pltpu.VMEM((L,), jnp.float32),
        pltpu.VMEM((L,), jnp.int32),
    ],
    compiler_params=pltpu.CompilerParams(needs_layout_passes=False),
)(_sc_index_body)

TM = 128


def _tc_scan_body(p_ref, x_ref, o_ref, carry_ref):
    j = pl.program_id(0)

    @pl.when(j == 0)
    def _():
        carry_ref[...] = x_ref[:, 0, :]

    i0 = lax.broadcasted_iota(jnp.int32, (TM, TM), 0)
    i1 = lax.broadcasted_iota(jnp.int32, (TM, TM), 1)
    eye = (i0 == i1).astype(jnp.float32)
    upper_incl = (i0 <= i1).astype(jnp.float32)
    tri = i0 >= i1

    for b in range(B):
        prow = p_ref[b : b + 1, :]
        la_row = jnp.log(1.0 - prow)
        c_row = lax.dot_general(
            la_row, upper_incl, (((1,), (0,)), ((), ())),
            preferred_element_type=jnp.float32,
            precision=lax.Precision.HIGHEST,
        )
        c_col = lax.dot_general(
            eye, c_row, (((1,), (1,)), ((), ())),
            preferred_element_type=jnp.float32,
            precision=lax.Precision.HIGHEST,
        )
        p_col = lax.dot_general(
            eye, prow, (((1,), (1,)), ((), ())),
            preferred_element_type=jnp.float32,
            precision=lax.Precision.HIGHEST,
        )
        Lmat = jnp.where(tri, jnp.exp(c_col - c_row), 0.0)
        dcol = jnp.exp(c_col)
        PX = p_col * x_ref[b]
        out_b = (
            lax.dot_general(
                Lmat, PX, (((1,), (0,)), ((), ())),
                preferred_element_type=jnp.float32,
                precision=lax.Precision.HIGHEST,
            )
            + dcol * carry_ref[b : b + 1, :]
        )
        o_ref[b] = out_b
        carry_ref[b : b + 1, :] = out_b[TM - 1 : TM, :]


def _tc_scan(p_chunked, chunked_states):
    return pl.pallas_call(
        _tc_scan_body,
        grid=(M // TM,),
        in_specs=[
            pl.BlockSpec((B, TM), lambda j: (0, j)),
            pl.BlockSpec((B, TM, D), lambda j: (0, j, 0)),
        ],
        out_specs=pl.BlockSpec((B, TM, D), lambda j: (0, j, 0)),
        out_shape=jax.ShapeDtypeStruct((B, M, D), jnp.float32),
        scratch_shapes=[pltpu.VMEM((B, D), jnp.float32)],
        compiler_params=pltpu.CompilerParams(
            dimension_semantics=("arbitrary",),
        ),
    )(p_chunked, chunked_states)


_RPW = (B * L) // _NW
_CH = 16
_NCH = _RPW // _CH


def _sc_gather_body(tab_hbm, idx_hbm, out_hbm, idx_v, buf_v, gsem, wsem):
    wid = lax.axis_index("s") * _NC + lax.axis_index("c")
    base = wid * _RPW
    pltpu.sync_copy(idx_hbm.at[pl.ds(base, _RPW)], idx_v)

    def gather(i, bsel):
        ivec = idx_v[pl.ds(i * _CH, _CH)]
        pltpu.async_copy(tab_hbm.at[ivec], buf_v.at[pl.ds(bsel * _CH, _CH)], gsem)

    def write(i, bsel):
        pltpu.async_copy(
            buf_v.at[pl.ds(bsel * _CH, _CH)], out_hbm.at[pl.ds(base + i * _CH, _CH)], wsem
        )

    def wait_gather():
        pltpu.make_async_copy(
            out_hbm.at[pl.ds(base, _CH)], buf_v.at[pl.ds(0, _CH)], gsem
        ).wait()

    def wait_write():
        pltpu.make_async_copy(
            buf_v.at[pl.ds(0, _CH)], out_hbm.at[pl.ds(base, _CH)], wsem
        ).wait()

    gather(0, 0)

    def body(i, _):
        bsel = jnp.remainder(i, 2)

        @pl.when(i >= 2)
        def _():
            wait_write()

        gather(i, bsel)
        wait_gather()
        write(i - 1, 1 - bsel)
        return 0

    lax.fori_loop(1, _NCH, body, 0)
    last = _NCH - 1
    wait_gather()
    write(last, jnp.remainder(last, 2))
    wait_write()
    wait_write()


_sc_gather = functools.partial(
    pl.kernel,
    out_type=jax.ShapeDtypeStruct((B * L, D), jnp.float32),
    mesh=_MESH,
    scratch_types=[
        pltpu.VMEM((_RPW,), jnp.int32),
        pltpu.VMEM((2 * _CH, D), jnp.float32),
        pltpu.SemaphoreType.DMA,
        pltpu.SemaphoreType.DMA,
    ],
    compiler_params=pltpu.CompilerParams(needs_layout_passes=False),
)(_sc_gather_body)


def kernel(chunked_states, boundary_mask, boundary_prob):
    mask_f32 = boundary_mask.astype(jnp.float32)
    pvals = boundary_prob[..., 1].astype(jnp.float32)
    cumT, nT = _tc_cumsum(mask_f32)
    p_chunked, gidx = _sc_index(mask_f32, pvals, cumT, nT)
    expanded = _tc_scan(p_chunked, chunked_states)
    out = _sc_gather(expanded.reshape(B * M, D), gidx.reshape(B * L))
    return out.reshape(B, L, D)


# speedup vs baseline: 23.9180x; 1.1514x over previous
import functools

import jax
import jax.numpy as jnp
from jax import lax
from jax.experimental import pallas as pl
from jax.experimental.pallas import tpu as pltpu
from jax.experimental.pallas import tpu_sc as plsc

B = 8
L = 2048
M = 1024
D = 2304

_MESH = plsc.VectorSubcoreMesh(core_axis_name="c", subcore_axis_name="s")
_NC = 2
_NW = 32


def _tc_cumsum_body(m_ref, cum_ref, nt_ref):
    i0 = lax.broadcasted_iota(jnp.int32, (128, 128), 0)
    i1 = lax.broadcasted_iota(jnp.int32, (128, 128), 1)
    U = (i0 <= i1).astype(jnp.float32)
    carry = jnp.zeros((B, 1), jnp.float32)
    for j in range(L // 128):
        blk = m_ref[:, j * 128 : (j + 1) * 128]
        loc = (
            lax.dot_general(
                blk, U, (((1,), (0,)), ((), ())),
                preferred_element_type=jnp.float32,
                precision=lax.Precision.HIGHEST,
            )
            + carry
        )
        cum_ref[:, j * 128 : (j + 1) * 128] = loc.astype(jnp.int32)
        carry = loc[:, 127:128]
    nt_ref[...] = jnp.broadcast_to(carry, (B, 16))


def _tc_cumsum(mask_f32):
    return pl.pallas_call(
        _tc_cumsum_body,
        out_shape=(
            jax.ShapeDtypeStruct((B, L), jnp.int32),
            jax.ShapeDtypeStruct((B, 16), jnp.float32),
        ),
    )(mask_f32)


def _sc_index_body(
    mask_hbm, p_hbm, cum_hbm, nt_hbm, pch_hbm, gidx_hbm,
    mask_v, p_v, cum_v, nt_v, pch_v, gidx_v,
):
    wid = lax.axis_index("s") * _NC + lax.axis_index("c")

    @pl.when(wid < B)
    def _():
        b = wid
        pltpu.sync_copy(mask_hbm.at[b], mask_v)
        pltpu.sync_copy(p_hbm.at[b], p_v)
        pltpu.sync_copy(cum_hbm.at[b], cum_v)
        pltpu.sync_copy(nt_hbm.at[b], nt_v)
        nT = nt_v[pl.ds(0, 16)].astype(jnp.int32)

        def body(i, _):
            mvec = mask_v[pl.ds(i * 16, 16)]
            pvec = p_v[pl.ds(i * 16, 16)]
            cumT = cum_v[pl.ds(i * 16, 16)]
            lidx = i * 16 + lax.iota(jnp.int32, 16)
            rank = jnp.where(mvec > 0.5, cumT - 1, nT + lidx - cumT)
            pc = jnp.clip(pvec, 1e-4, 1.0 - 1e-4)
            plsc.store_scatter(pch_v, [rank], pc)
            gidx_v[pl.ds(i * 16, 16)] = b * M + jnp.clip(cumT - 1, 0, M - 1)
            return _

        lax.fori_loop(0, L // 16, body, jnp.int32(0))
        pltpu.sync_copy(pch_v.at[pl.ds(0, M)], pch_hbm.at[b])
        pltpu.sync_copy(gidx_v, gidx_hbm.at[b])


_sc_index = functools.partial(
    pl.kernel,
    out_type=(
        jax.ShapeDtypeStruct((B, M), jnp.float32),
        jax.ShapeDtypeStruct((B, L), jnp.int32),
    ),
    mesh=_MESH,
    scratch_types=[
        pltpu.VMEM((L,), jnp.float32),
        pltpu.VMEM((L,), jnp.float32),
        pltpu.VMEM((L,), jnp.int32),
        pltpu.VMEM((16,), jnp.float32),
        pltpu.VMEM((L,), jnp.float32),
        pltpu.VMEM((L,), jnp.int32),
    ],
    compiler_params=pltpu.CompilerParams(needs_layout_passes=False),
)(_sc_index_body)

TM = 128
SUB = 64


def _tc_scan_body(p_ref, x_ref, o_ref, carry_ref):
    j = pl.program_id(0)

    @pl.when(j == 0)
    def _():
        carry_ref[...] = x_ref[:, 0, :]

    i0 = lax.broadcasted_iota(jnp.int32, (SUB, SUB), 0)
    i1 = lax.broadcasted_iota(jnp.int32, (SUB, SUB), 1)
    eye = (i0 == i1).astype(jnp.float32)
    upper_incl = (i0 <= i1).astype(jnp.float32)
    tri = i0 >= i1

    for b in range(B):
        carry = carry_ref[b : b + 1, :]
        for s in range(TM // SUB):
            sl = slice(s * SUB, (s + 1) * SUB)
            prow = p_ref[b : b + 1, sl]
            la_row = jnp.log(1.0 - prow)
            c_row = lax.dot_general(
                la_row, upper_incl, (((1,), (0,)), ((), ())),
                preferred_element_type=jnp.float32,
                precision=lax.Precision.HIGHEST,
            )
            c_col = lax.dot_general(
                eye, c_row, (((1,), (1,)), ((), ())),
                preferred_element_type=jnp.float32,
                precision=lax.Precision.HIGHEST,
            )
            p_col = lax.dot_general(
                eye, prow, (((1,), (1,)), ((), ())),
                preferred_element_type=jnp.float32,
                precision=lax.Precision.HIGHEST,
            )
            Lmat = jnp.where(tri, jnp.exp(c_col - c_row), 0.0)
            dcol = jnp.exp(c_col)
            PX = p_col * x_ref[b, sl, :]
            out_b = (
                lax.dot_general(
                    Lmat, PX, (((1,), (0,)), ((), ())),
                    preferred_element_type=jnp.float32,
                    precision=lax.Precision.DEFAULT,
                )
                + dcol * carry
            )
            o_ref[b, sl, :] = out_b
            carry = out_b[SUB - 1 : SUB, :]
        carry_ref[b : b + 1, :] = carry


def _tc_scan(p_chunked, chunked_states):
    return pl.pallas_call(
        _tc_scan_body,
        grid=(M // TM,),
        in_specs=[
            pl.BlockSpec((B, TM), lambda j: (0, j)),
            pl.BlockSpec((B, TM, D), lambda j: (0, j, 0)),
        ],
        out_specs=pl.BlockSpec((B, TM, D), lambda j: (0, j, 0)),
        out_shape=jax.ShapeDtypeStruct((B, M, D), jnp.float32),
        scratch_shapes=[pltpu.VMEM((B, D), jnp.float32)],
        compiler_params=pltpu.CompilerParams(
            dimension_semantics=("arbitrary",),
        ),
    )(p_chunked, chunked_states)


_RPW = (B * L) // _NW
_CH = 16
_NCH = _RPW // _CH


def _sc_gather_body(tab_hbm, idx_hbm, out_hbm, idx_v, buf_v, gsem, wsem):
    wid = lax.axis_index("s") * _NC + lax.axis_index("c")
    base = wid * _RPW
    pltpu.sync_copy(idx_hbm.at[pl.ds(base, _RPW)], idx_v)

    def gather(i, bsel):
        ivec = idx_v[pl.ds(i * _CH, _CH)]
        pltpu.async_copy(tab_hbm.at[ivec], buf_v.at[pl.ds(bsel * _CH, _CH)], gsem)

    def write(i, bsel):
        pltpu.async_copy(
            buf_v.at[pl.ds(bsel * _CH, _CH)], out_hbm.at[pl.ds(base + i * _CH, _CH)], wsem
        )

    def wait_gather():
        pltpu.make_async_copy(
            out_hbm.at[pl.ds(base, _CH)], buf_v.at[pl.ds(0, _CH)], gsem
        ).wait()

    def wait_write():
        pltpu.make_async_copy(
            buf_v.at[pl.ds(0, _CH)], out_hbm.at[pl.ds(base, _CH)], wsem
        ).wait()

    gather(0, 0)

    def body(i, _):
        bsel = jnp.remainder(i, 2)

        @pl.when(i >= 2)
        def _():
            wait_write()

        gather(i, bsel)
        wait_gather()
        write(i - 1, 1 - bsel)
        return 0

    lax.fori_loop(1, _NCH, body, 0)
    last = _NCH - 1
    wait_gather()
    write(last, jnp.remainder(last, 2))
    wait_write()
    wait_write()


_sc_gather = functools.partial(
    pl.kernel,
    out_type=jax.ShapeDtypeStruct((B * L, D), jnp.float32),
    mesh=_MESH,
    scratch_types=[
        pltpu.VMEM((_RPW,), jnp.int32),
        pltpu.VMEM((2 * _CH, D), jnp.float32),
        pltpu.SemaphoreType.DMA,
        pltpu.SemaphoreType.DMA,
    ],
    compiler_params=pltpu.CompilerParams(needs_layout_passes=False),
)(_sc_gather_body)


def kernel(chunked_states, boundary_mask, boundary_prob):
    mask_f32 = boundary_mask.astype(jnp.float32)
    pvals = boundary_prob[..., 1].astype(jnp.float32)
    cumT, nT = _tc_cumsum(mask_f32)
    p_chunked, gidx = _sc_index(mask_f32, pvals, cumT, nT)
    expanded = _tc_scan(p_chunked, chunked_states)
    out = _sc_gather(expanded.reshape(B * M, D), gidx.reshape(B * L))
    return out.reshape(B, L, D)


# speedup vs baseline: 24.0823x; 1.0069x over previous
import functools

import jax
import jax.numpy as jnp
from jax import lax
from jax.experimental import pallas as pl
from jax.experimental.pallas import tpu as pltpu
from jax.experimental.pallas import tpu_sc as plsc

B = 8
L = 2048
M = 1024
D = 2304

_MESH = plsc.VectorSubcoreMesh(core_axis_name="c", subcore_axis_name="s")
_NC = 2
_NW = 32


def _sc_index_body(
    mask_hbm, p_hbm, pch_hbm, gidx_hbm,
    mask_v, p_v, pch_v, gidx_v,
):
    wid = lax.axis_index("s") * _NC + lax.axis_index("c")

    @pl.when(wid < B)
    def _():
        b = wid
        pltpu.sync_copy(mask_hbm.at[b], mask_v)
        pltpu.sync_copy(p_hbm.at[b], p_v)

        def count_body(i, acc):
            return acc + jnp.sum(mask_v[pl.ds(i * 16, 16)], axis=0)

        nT_f = lax.fori_loop(0, L // 16, count_body, jnp.float32(0.0))
        nT = nT_f.astype(jnp.int32)

        def body(i, s):
            mvec = mask_v[pl.ds(i * 16, 16)]
            pvec = p_v[pl.ds(i * 16, 16)]
            cumT = (s + plsc.cumsum(mvec)).astype(jnp.int32)
            lidx = i * 16 + lax.iota(jnp.int32, 16)
            rank = jnp.where(mvec > 0.5, cumT - 1, nT + lidx - cumT)
            pc = jnp.clip(pvec, 1e-4, 1.0 - 1e-4)
            plsc.store_scatter(pch_v, [rank], pc)
            gidx_v[pl.ds(i * 16, 16)] = b * M + jnp.clip(cumT - 1, 0, M - 1)
            return s + jnp.sum(mvec, axis=0)

        lax.fori_loop(0, L // 16, body, jnp.float32(0.0))
        pltpu.sync_copy(pch_v.at[pl.ds(0, M)], pch_hbm.at[b])
        pltpu.sync_copy(gidx_v, gidx_hbm.at[b])


_sc_index = functools.partial(
    pl.kernel,
    out_type=(
        jax.ShapeDtypeStruct((B, M), jnp.float32),
        jax.ShapeDtypeStruct((B, L), jnp.int32),
    ),
    mesh=_MESH,
    scratch_types=[
        pltpu.VMEM((L,), jnp.float32),
        pltpu.VMEM((L,), jnp.float32),
        pltpu.VMEM((L,), jnp.float32),
        pltpu.VMEM((L,), jnp.int32),
    ],
    compiler_params=pltpu.CompilerParams(needs_layout_passes=False),
)(_sc_index_body)

TM = 128
SUB = 64


def _tc_scan_body(p_ref, x_ref, o_ref, carry_ref):
    j = pl.program_id(0)

    @pl.when(j == 0)
    def _():
        carry_ref[...] = x_ref[:, 0, :]

    i0 = lax.broadcasted_iota(jnp.int32, (SUB, SUB), 0)
    i1 = lax.broadcasted_iota(jnp.int32, (SUB, SUB), 1)
    eye = (i0 == i1).astype(jnp.float32)
    upper_incl = (i0 <= i1).astype(jnp.float32)
    tri = i0 >= i1

    for b in range(B):
        carry = carry_ref[b : b + 1, :]
        for s in range(TM // SUB):
            sl = slice(s * SUB, (s + 1) * SUB)
            prow = p_ref[b : b + 1, sl]
            la_row = jnp.log(1.0 - prow)
            c_row = lax.dot_general(
                la_row, upper_incl, (((1,), (0,)), ((), ())),
                preferred_element_type=jnp.float32,
                precision=lax.Precision.HIGHEST,
            )
            c_col = lax.dot_general(
                eye, c_row, (((1,), (1,)), ((), ())),
                preferred_element_type=jnp.float32,
                precision=lax.Precision.HIGHEST,
            )
            p_col = lax.dot_general(
                eye, prow, (((1,), (1,)), ((), ())),
                preferred_element_type=jnp.float32,
                precision=lax.Precision.HIGHEST,
            )
            Lmat = jnp.where(tri, jnp.exp(c_col - c_row), 0.0)
            dcol = jnp.exp(c_col)
            PX = p_col * x_ref[b, sl, :]
            out_b = (
                lax.dot_general(
                    Lmat, PX, (((1,), (0,)), ((), ())),
                    preferred_element_type=jnp.float32,
                    precision=lax.Precision.DEFAULT,
                )
                + dcol * carry
            )
            o_ref[b, sl, :] = out_b
            carry = out_b[SUB - 1 : SUB, :]
        carry_ref[b : b + 1, :] = carry


def _tc_scan(p_chunked, chunked_states):
    return pl.pallas_call(
        _tc_scan_body,
        grid=(M // TM,),
        in_specs=[
            pl.BlockSpec((B, TM), lambda j: (0, j)),
            pl.BlockSpec((B, TM, D), lambda j: (0, j, 0)),
        ],
        out_specs=pl.BlockSpec((B, TM, D), lambda j: (0, j, 0)),
        out_shape=jax.ShapeDtypeStruct((B, M, D), jnp.float32),
        scratch_shapes=[pltpu.VMEM((B, D), jnp.float32)],
        compiler_params=pltpu.CompilerParams(
            dimension_semantics=("arbitrary",),
        ),
    )(p_chunked, chunked_states)


_RPW = (B * L) // _NW
_CH = 16
_NCH = _RPW // _CH


def _sc_gather_body(tab_hbm, idx_hbm, out_hbm, idx_v, buf_v, gsem, wsem):
    wid = lax.axis_index("s") * _NC + lax.axis_index("c")
    base = wid * _RPW
    pltpu.sync_copy(idx_hbm.at[pl.ds(base, _RPW)], idx_v)

    def gather(i, bsel):
        ivec = idx_v[pl.ds(i * _CH, _CH)]
        pltpu.async_copy(tab_hbm.at[ivec], buf_v.at[pl.ds(bsel * _CH, _CH)], gsem)

    def write(i, bsel):
        pltpu.async_copy(
            buf_v.at[pl.ds(bsel * _CH, _CH)], out_hbm.at[pl.ds(base + i * _CH, _CH)], wsem
        )

    def wait_gather():
        pltpu.make_async_copy(
            out_hbm.at[pl.ds(base, _CH)], buf_v.at[pl.ds(0, _CH)], gsem
        ).wait()

    def wait_write():
        pltpu.make_async_copy(
            buf_v.at[pl.ds(0, _CH)], out_hbm.at[pl.ds(base, _CH)], wsem
        ).wait()

    gather(0, 0)
    gather(1, 1)

    def body(i, _):
        @pl.when(i >= 3)
        def _():
            wait_write()

        gather(i, jnp.remainder(i, 3))
        wait_gather()
        write(i - 2, jnp.remainder(i + 1, 3))
        return 0

    lax.fori_loop(2, _NCH, body, 0)
    last = _NCH - 1
    wait_gather()
    wait_write()
    write(last - 1, jnp.remainder(last - 1, 3))
    wait_gather()
    write(last, jnp.remainder(last, 3))
    wait_write()
    wait_write()


_sc_gather = functools.partial(
    pl.kernel,
    out_type=jax.ShapeDtypeStruct((B * L, D), jnp.float32),
    mesh=_MESH,
    scratch_types=[
        pltpu.VMEM((_RPW,), jnp.int32),
        pltpu.VMEM((3 * _CH, D), jnp.float32),
        pltpu.SemaphoreType.DMA,
        pltpu.SemaphoreType.DMA,
    ],
    compiler_params=pltpu.CompilerParams(needs_layout_passes=False),
)(_sc_gather_body)


def kernel(chunked_states, boundary_mask, boundary_prob):
    mask_f32 = boundary_mask.astype(jnp.float32)
    pvals = boundary_prob[..., 1].astype(jnp.float32)
    p_chunked, gidx = _sc_index(mask_f32, pvals)
    expanded = _tc_scan(p_chunked, chunked_states)
    out = _sc_gather(expanded.reshape(B * M, D), gidx.reshape(B * L))
    return out.reshape(B, L, D)


# speedup vs baseline: 24.6062x; 1.0218x over previous
import functools

import jax
import jax.numpy as jnp
from jax import lax
from jax.experimental import pallas as pl
from jax.experimental.pallas import tpu as pltpu
from jax.experimental.pallas import tpu_sc as plsc

B = 8
L = 2048
M = 1024
D = 2304

_MESH = plsc.VectorSubcoreMesh(core_axis_name="c", subcore_axis_name="s")
_NC = 2
_NW = 32


def _sc_index_body(
    mask_hbm, p_hbm, pch_hbm, gidx_hbm,
    mask_v, p_v, pch_v, gidx_v,
):
    wid = lax.axis_index("s") * _NC + lax.axis_index("c")

    @pl.when(wid < B)
    def _():
        b = wid
        pltpu.sync_copy(mask_hbm.at[b], mask_v)
        pltpu.sync_copy(p_hbm.at[b], p_v)

        def count_body(i, acc):
            return acc + jnp.sum(mask_v[pl.ds(i * 16, 16)], axis=0)

        nT_f = lax.fori_loop(0, L // 16, count_body, jnp.float32(0.0))
        nT = nT_f.astype(jnp.int32)

        def body(i, s):
            mvec = mask_v[pl.ds(i * 16, 16)]
            pvec = p_v[pl.ds(i * 16, 16)]
            cumT = (s + plsc.cumsum(mvec)).astype(jnp.int32)
            lidx = i * 16 + lax.iota(jnp.int32, 16)
            rank = jnp.where(mvec > 0.5, cumT - 1, nT + lidx - cumT)
            pc = jnp.clip(pvec, 1e-4, 1.0 - 1e-4)
            plsc.store_scatter(pch_v, [rank], pc)
            gidx_v[pl.ds(i * 16, 16)] = b * M + jnp.clip(cumT - 1, 0, M - 1)
            return s + jnp.sum(mvec, axis=0)

        lax.fori_loop(0, L // 16, body, jnp.float32(0.0))
        pltpu.sync_copy(pch_v.at[pl.ds(0, M)], pch_hbm.at[b])
        pltpu.sync_copy(gidx_v, gidx_hbm.at[b])


_sc_index = functools.partial(
    pl.kernel,
    out_type=(
        jax.ShapeDtypeStruct((B, M), jnp.float32),
        jax.ShapeDtypeStruct((B, L), jnp.int32),
    ),
    mesh=_MESH,
    scratch_types=[
        pltpu.VMEM((L,), jnp.float32),
        pltpu.VMEM((L,), jnp.float32),
        pltpu.VMEM((L,), jnp.float32),
        pltpu.VMEM((L,), jnp.int32),
    ],
    compiler_params=pltpu.CompilerParams(needs_layout_passes=False),
)(_sc_index_body)

TM = 128
SUB = 128


def _tc_scan_body(p_ref, x_ref, o_ref, carry_ref):
    j = pl.program_id(0)

    @pl.when(j == 0)
    def _():
        carry_ref[...] = x_ref[:, 0, :]

    i0 = lax.broadcasted_iota(jnp.int32, (SUB, SUB), 0)
    i1 = lax.broadcasted_iota(jnp.int32, (SUB, SUB), 1)
    eye = (i0 == i1).astype(jnp.float32)
    upper_incl = (i0 <= i1).astype(jnp.float32)
    tri = i0 >= i1

    for b in range(B):
        carry = carry_ref[b : b + 1, :]
        for s in range(TM // SUB):
            sl = slice(s * SUB, (s + 1) * SUB)
            prow = p_ref[b : b + 1, sl]
            la_row = jnp.log(1.0 - prow)
            c_row = lax.dot_general(
                la_row, upper_incl, (((1,), (0,)), ((), ())),
                preferred_element_type=jnp.float32,
                precision=lax.Precision.HIGHEST,
            )
            c_col = lax.dot_general(
                eye, c_row, (((1,), (1,)), ((), ())),
                preferred_element_type=jnp.float32,
                precision=lax.Precision.HIGHEST,
            )
            p_col = lax.dot_general(
                eye, prow, (((1,), (1,)), ((), ())),
                preferred_element_type=jnp.float32,
                precision=lax.Precision.HIGHEST,
            )
            Lmat = jnp.where(tri, jnp.exp(c_col - c_row), 0.0)
            dcol = jnp.exp(c_col)
            PX = p_col * x_ref[b, sl, :]
            out_b = (
                lax.dot_general(
                    Lmat, PX, (((1,), (0,)), ((), ())),
                    preferred_element_type=jnp.float32,
                    precision=lax.Precision.DEFAULT,
                )
                + dcol * carry
            )
            o_ref[b, sl, :] = out_b
            carry = out_b[SUB - 1 : SUB, :]
        carry_ref[b : b + 1, :] = carry


def _tc_scan(p_chunked, chunked_states):
    return pl.pallas_call(
        _tc_scan_body,
        grid=(M // TM,),
        in_specs=[
            pl.BlockSpec((B, TM), lambda j: (0, j)),
            pl.BlockSpec((B, TM, D), lambda j: (0, j, 0)),
        ],
        out_specs=pl.BlockSpec((B, TM, D), lambda j: (0, j, 0)),
        out_shape=jax.ShapeDtypeStruct((B, M, D), jnp.float32),
        scratch_shapes=[pltpu.VMEM((B, D), jnp.float32)],
        compiler_params=pltpu.CompilerParams(
            dimension_semantics=("arbitrary",),
        ),
    )(p_chunked, chunked_states)


_RPW = (B * L) // _NW
_CH = 16
_NCH = _RPW // _CH


def _sc_gather_body(tab_hbm, idx_hbm, out_hbm, idx_v, buf_v, gsem, wsem):
    wid = lax.axis_index("s") * _NC + lax.axis_index("c")
    base = wid * _RPW
    pltpu.sync_copy(idx_hbm.at[pl.ds(base, _RPW)], idx_v)

    def gather(i, bsel):
        ivec = idx_v[pl.ds(i * _CH, _CH)]
        pltpu.async_copy(tab_hbm.at[ivec], buf_v.at[pl.ds(bsel * _CH, _CH)], gsem)

    def write(i, bsel):
        pltpu.async_copy(
            buf_v.at[pl.ds(bsel * _CH, _CH)], out_hbm.at[pl.ds(base + i * _CH, _CH)], wsem
        )

    def wait_gather():
        pltpu.make_async_copy(
            out_hbm.at[pl.ds(base, _CH)], buf_v.at[pl.ds(0, _CH)], gsem
        ).wait()

    def wait_write():
        pltpu.make_async_copy(
            buf_v.at[pl.ds(0, _CH)], out_hbm.at[pl.ds(base, _CH)], wsem
        ).wait()

    gather(0, 0)
    gather(1, 1)

    def body(i, _):
        @pl.when(i >= 3)
        def _():
            wait_write()

        gather(i, jnp.remainder(i, 3))
        wait_gather()
        write(i - 2, jnp.remainder(i + 1, 3))
        return 0

    lax.fori_loop(2, _NCH, body, 0)
    last = _NCH - 1
    wait_gather()
    wait_write()
    write(last - 1, jnp.remainder(last - 1, 3))
    wait_gather()
    write(last, jnp.remainder(last, 3))
    wait_write()
    wait_write()


_sc_gather = functools.partial(
    pl.kernel,
    out_type=jax.ShapeDtypeStruct((B * L, D), jnp.float32),
    mesh=_MESH,
    scratch_types=[
        pltpu.VMEM((_RPW,), jnp.int32),
        pltpu.VMEM((3 * _CH, D), jnp.float32),
        pltpu.SemaphoreType.DMA,
        pltpu.SemaphoreType.DMA,
    ],
    compiler_params=pltpu.CompilerParams(needs_layout_passes=False),
)(_sc_gather_body)


def kernel(chunked_states, boundary_mask, boundary_prob):
    mask_f32 = boundary_mask.astype(jnp.float32)
    pvals = boundary_prob[..., 1].astype(jnp.float32)
    p_chunked, gidx = _sc_index(mask_f32, pvals)
    expanded = _tc_scan(p_chunked, chunked_states)
    out = _sc_gather(expanded.reshape(B * M, D), gidx.reshape(B * L))
    return out.reshape(B, L, D)
